# Initial kernel scaffold; baseline (speedup 1.0000x reference)
import jax
import jax.numpy as jnp
from jax.experimental import pallas as pl


def kernel(x, indices_spa, indices_sph):
    raise NotImplementedError("write your pallas kernel here")


# speedup vs baseline: 63.3853x; 63.3853x over previous
import jax
import jax.numpy as jnp
from jax import lax
from jax.experimental import pallas as pl

_V = 192
_KS = 4
_OUTV = _V * _KS
_S = 64
_Q = 512


def _unpool_body(x_ref, sph_ref, spa_ref, o_ref):
    xs = x_ref[0, 0]
    sph = sph_ref[0, 0]
    spa = spa_ref[0, 0]

    spht = sph.T
    spht4 = jnp.broadcast_to(spht[:, None, :], (_V, _KS, _S)).reshape(_OUTV, _S)
    xs4 = jnp.broadcast_to(xs[:, None, :], (_V, _KS, _S)).reshape(_OUTV, _S)
    u = lax.broadcasted_iota(jnp.int32, (_OUTV, _S), 0)
    xm = jnp.where(spht4 == u, xs4, 0.0)

    qi = lax.broadcasted_iota(jnp.int32, (_S, _Q), 1)
    s_of_q = 16 * (qi // 128) + 4 * ((qi // 16) % 4) + (qi // 2) % 4
    si = lax.broadcasted_iota(jnp.int32, (_S, _Q), 0)
    e = (s_of_q == si).astype(jnp.float32)

    stacked = jnp.concatenate([xm, spa.astype(jnp.float32)], axis=0)
    up = jnp.dot(stacked, e, preferred_element_type=jnp.float32,
                 precision=lax.Precision.HIGHEST)
    xm_up = up[:_OUTV]
    spa_up = up[_OUTV:]
    qf = lax.broadcasted_iota(jnp.int32, (_OUTV, _Q), 1).astype(jnp.float32)
    o_ref[0, 0] = jnp.where(spa_up == qf, xm_up, 0.0)


def kernel(x, indices_spa, indices_sph, *, interpret=False):
    b, fin, v, xx, yy, zz = x.shape
    xr = x.reshape(b, fin, _V, _S)
    sphr = indices_sph.reshape(b, fin, _S, _V)
    spar = indices_spa.reshape(b, fin, _OUTV, _S)
    out = pl.pallas_call(
        _unpool_body,
        grid=(b, fin),
        in_specs=[
            pl.BlockSpec((1, 1, _V, _S), lambda i, j: (i, j, 0, 0)),
            pl.BlockSpec((1, 1, _S, _V), lambda i, j: (i, j, 0, 0)),
            pl.BlockSpec((1, 1, _OUTV, _S), lambda i, j: (i, j, 0, 0)),
        ],
        out_specs=pl.BlockSpec((1, 1, _OUTV, _Q), lambda i, j: (i, j, 0, 0)),
        out_shape=jax.ShapeDtypeStruct((b, fin, _OUTV, _Q), x.dtype),
        interpret=interpret,
    )(xr, sphr, spar)
    return out.reshape(b, fin, _OUTV, 2 * xx, 2 * yy, 2 * zz)


# speedup vs baseline: 91.9696x; 1.4510x over previous
import jax
import jax.numpy as jnp
from jax import lax
from jax.experimental import pallas as pl

_V = 192
_KS = 4
_OUTV = _V * _KS
_S = 64
_Q = 512


def _unpool_body(x_ref, sph_ref, spa_ref, o_ref):
    xs = x_ref[0, 0]
    sph = sph_ref[0, 0]
    spa = spa_ref[0, 0]

    spht = sph.T
    spht4 = jnp.broadcast_to(spht[:, None, :], (_V, _KS, _S)).reshape(_OUTV, _S)
    xs4 = jnp.broadcast_to(xs[:, None, :], (_V, _KS, _S)).reshape(_OUTV, _S)
    u = lax.broadcasted_iota(jnp.int32, (_OUTV, _S), 0)
    xm = jnp.where(spht4 == u, xs4, 0.0)

    qi = lax.broadcasted_iota(jnp.int32, (_S, _Q), 1)
    s_of_q = 16 * (qi // 128) + 4 * ((qi // 16) % 4) + (qi // 2) % 4
    si = lax.broadcasted_iota(jnp.int32, (_S, _Q), 0)
    e = (s_of_q == si).astype(jnp.float32)

    sij = lax.broadcasted_iota(jnp.int32, (_OUTV, _S), 1)
    base = 128 * (sij // 16) + 16 * ((sij // 4) % 4) + 2 * (sij % 4)
    t = (spa - base).astype(jnp.float32)

    hi = xm.astype(jnp.bfloat16).astype(jnp.float32)
    lo = xm - hi
    stacked = jnp.concatenate([hi, lo, t], axis=0)
    up = jnp.dot(stacked, e, preferred_element_type=jnp.float32)
    xm_up = up[:_OUTV] + up[_OUTV:2 * _OUTV]
    t_up = up[2 * _OUTV:]
    qj = lax.broadcasted_iota(jnp.int32, (_OUTV, _Q), 1)
    qoff = (64 * ((qj // 64) % 2) + 8 * ((qj // 8) % 2) + (qj % 2)).astype(jnp.float32)
    o_ref[0, 0] = jnp.where(t_up == qoff, xm_up, 0.0)


def kernel(x, indices_spa, indices_sph, *, interpret=False):
    b, fin, v, xx, yy, zz = x.shape
    xr = x.reshape(b, fin, _V, _S)
    sphr = indices_sph.reshape(b, fin, _S, _V)
    spar = indices_spa.reshape(b, fin, _OUTV, _S)
    out = pl.pallas_call(
        _unpool_body,
        grid=(b, fin),
        in_specs=[
            pl.BlockSpec((1, 1, _V, _S), lambda i, j: (i, j, 0, 0)),
            pl.BlockSpec((1, 1, _S, _V), lambda i, j: (i, j, 0, 0)),
            pl.BlockSpec((1, 1, _OUTV, _S), lambda i, j: (i, j, 0, 0)),
        ],
        out_specs=pl.BlockSpec((1, 1, _OUTV, _Q), lambda i, j: (i, j, 0, 0)),
        out_shape=jax.ShapeDtypeStruct((b, fin, _OUTV, _Q), x.dtype),
        interpret=interpret,
    )(xr, sphr, spar)
    return out.reshape(b, fin, _OUTV, 2 * xx, 2 * yy, 2 * zz)
